# baseline (speedup 1.0000x reference)
import functools
import math

import jax
import jax.numpy as jnp
from jax import lax
from jax.experimental import pallas as pl
from jax.experimental.pallas import tpu as pltpu
from jax.experimental.pallas import tpu_sc as plsc

VOCAB = 100000
D = 128
B = 4
T = 2048
NC, NS, L = 2, 16, 16
NW = NC * NS
PW = T // NW
NQ = 4
QW = PW // NQ
QR = B * QW
SCALE = math.sqrt(D)

_mesh = plsc.VectorSubcoreMesh(core_axis_name="c", subcore_axis_name="s")


@functools.partial(
    pl.kernel,
    mesh=_mesh,
    out_type=jax.ShapeDtypeStruct((B, T, D), jnp.float32),
    scratch_types=[
        pltpu.VMEM((NQ, QR), jnp.int32),
        pltpu.VMEM((NQ * QR, D), jnp.float32),
        pltpu.VMEM((PW, D), jnp.float32),
        pltpu.SemaphoreType.DMA,
        pltpu.SemaphoreType.DMA,
    ]
    + [pltpu.SemaphoreType.DMA] * NQ
    + [pltpu.SemaphoreType.DMA],
)
def _embed(idx_hbm, tok_hbm, pos_hbm, out_hbm, idx_v, rows_v, pos_v,
           isem, psem, *rest):
    qsems, osem = rest[:NQ], rest[NQ]
    wid = lax.axis_index("s") * NC + lax.axis_index("c")
    p0 = wid * PW

    pcopy = pltpu.async_copy(pos_hbm.at[pl.ds(p0, PW)], pos_v, psem)
    pltpu.async_copy(idx_hbm.at[wid], idx_v, isem).wait()
    gathers = [
        pltpu.async_copy(
            tok_hbm.at[idx_v.at[q]],
            rows_v.at[pl.ds(q * QR, QR)], qsems[q])
        for q in range(NQ)
    ]

    out_waits = []
    for q, g in enumerate(gathers):
        g.wait()
        if q == 0:
            pcopy.wait()

        def body(i, carry, q=q):
            pi = q * QW + i
            for j in range(D // L):
                sl = pl.ds(j * L, L)
                pv = pos_v[pi, sl]
                for b in range(B):
                    row = q * QR + b * QW + i
                    rows_v[row, sl] = rows_v[row, sl] * SCALE + pv
            return carry

        lax.fori_loop(0, QW, body, 0)
        for b in range(B):
            out_waits.append(pltpu.async_copy(
                rows_v.at[pl.ds(q * QR + b * QW, QW)],
                out_hbm.at[b, pl.ds(p0 + q * QW, QW)], osem))

    for wt in out_waits:
        wt.wait()


def kernel(token_ids, tok_table, pos_table):
    idx = (token_ids.astype(jnp.int32)
           .reshape(B, NW, NQ, QW)
           .transpose(1, 2, 0, 3)
           .reshape(NW, NQ, QR))
    out = _embed(idx, tok_table, pos_table)
    return out


# speedup vs baseline: 1.0090x; 1.0090x over previous
import functools
import math

import jax
import jax.numpy as jnp
from jax import lax
from jax.experimental import pallas as pl
from jax.experimental.pallas import tpu as pltpu
from jax.experimental.pallas import tpu_sc as plsc

VOCAB = 100000
D = 128
B = 4
T = 2048
NC, NS, L = 2, 16, 16
NW = NC * NS
PW = T // NW
NQ = 2
QW = PW // NQ
QR = B * QW
SCALE = math.sqrt(D)

_mesh = plsc.VectorSubcoreMesh(core_axis_name="c", subcore_axis_name="s")


@functools.partial(
    pl.kernel,
    mesh=_mesh,
    out_type=jax.ShapeDtypeStruct((B, T, D), jnp.float32),
    scratch_types=[
        pltpu.VMEM((NQ, QR), jnp.int32),
        pltpu.VMEM((NQ * QR, D), jnp.float32),
        pltpu.VMEM((PW, D), jnp.float32),
        pltpu.SemaphoreType.DMA,
        pltpu.SemaphoreType.DMA,
    ]
    + [pltpu.SemaphoreType.DMA] * NQ
    + [pltpu.SemaphoreType.DMA],
)
def _embed(idx_hbm, tok_hbm, pos_hbm, out_hbm, idx_v, rows_v, pos_v,
           isem, psem, *rest):
    qsems, osem = rest[:NQ], rest[NQ]
    wid = lax.axis_index("s") * NC + lax.axis_index("c")
    p0 = wid * PW

    pcopy = pltpu.async_copy(pos_hbm.at[pl.ds(p0, PW)], pos_v, psem)
    pltpu.async_copy(idx_hbm.at[wid], idx_v, isem).wait()
    gathers = [
        pltpu.async_copy(
            tok_hbm.at[idx_v.at[q]],
            rows_v.at[pl.ds(q * QR, QR)], qsems[q])
        for q in range(NQ)
    ]

    out_waits = []
    for q, g in enumerate(gathers):
        g.wait()
        if q == 0:
            pcopy.wait()

        def body(i, carry, q=q):
            pi = q * QW + i
            for j in range(D // L):
                sl = pl.ds(j * L, L)
                pv = pos_v[pi, sl]
                for b in range(B):
                    row = q * QR + b * QW + i
                    rows_v[row, sl] = rows_v[row, sl] * SCALE + pv
            return carry

        lax.fori_loop(0, QW, body, 0)
        for b in range(B):
            out_waits.append(pltpu.async_copy(
                rows_v.at[pl.ds(q * QR + b * QW, QW)],
                out_hbm.at[b, pl.ds(p0 + q * QW, QW)], osem))

    for wt in out_waits:
        wt.wait()


def kernel(token_ids, tok_table, pos_table):
    idx = (token_ids.astype(jnp.int32)
           .reshape(B, NW, NQ, QW)
           .transpose(1, 2, 0, 3)
           .reshape(NW, NQ, QR))
    out = _embed(idx, tok_table, pos_table)
    return out
